# baseline (speedup 1.0000x reference)
import functools

import jax
import jax.numpy as jnp
import numpy as np
from jax.experimental import pallas as pl
from jax.sharding import Mesh, PartitionSpec as P

try:
    from jax.experimental.shard_map import shard_map
except ImportError:
    from jax import shard_map

NUM_HEADS = 4
NUM_STEPS = 3
SKIP = 0.3
DH = 32


def _dotT(a, b):
    return jax.lax.dot_general(
        a, b, (((1,), (1,)), ((), ())), preferred_element_type=jnp.float32
    )


def _dot(a, b):
    return jnp.dot(a, b, preferred_element_type=jnp.float32)


def _sglc_body(x_ref, sup_ref, wgl_ref, wmsg_ref, bmsg_ref, wrzh_ref,
               urz_ref, uh_ref, hmask_ref, out_ref, adj_ref):
    t = pl.program_id(0)
    nb, n, d = x_ref.shape[1], x_ref.shape[2], x_ref.shape[3]
    bf = jnp.bfloat16
    x_all = x_ref[0].reshape(nb * n, d)

    xp = _dot(x_all.astype(bf), wgl_ref[...])
    sq = (xp * xp).astype(bf)
    s = _dot(sq, hmask_ref[...])
    xn = (xp * (0.5 / (jnp.sqrt(s) + 1e-8))).astype(bf)

    @pl.when(t == 0)
    def _init():
        adj_ref[...] = sup_ref[...]

    adjs = []
    for b in range(nb):
        xnb = xn[b * n:(b + 1) * n]
        attn = jnp.maximum(_dotT(xnb, xnb), 0.0)
        learned = attn / (jnp.sum(attn, axis=-1, keepdims=True) + 1e-8)
        adj = SKIP * adj_ref[b] + (1.0 - SKIP) * learned
        adj_ref[b] = adj
        adjs.append(adj.astype(bf))

    wmsg = wmsg_ref[...]
    bmsg = bmsg_ref[...]
    wrzh = wrzh_ref[...]
    urz = urz_ref[...]
    uh = uh_ref[...]
    h = x_all
    for _ in range(NUM_STEPS):
        hb = h.astype(bf)
        a_all = jnp.concatenate(
            [_dot(adjs[b], hb[b * n:(b + 1) * n]) for b in range(nb)], axis=0)
        m = (_dot(a_all.astype(bf), wmsg) + bmsg).astype(bf)
        gates = _dot(m, wrzh)
        hu = _dot(hb, urz)
        r = jax.nn.sigmoid(gates[:, :d] + hu[:, :d])
        z = jax.nn.sigmoid(gates[:, d:2 * d] + hu[:, d:2 * d])
        q = _dot((r * h).astype(bf), uh)
        hh = jnp.tanh(gates[:, 2 * d:] + q)
        h = (1.0 - z) * h + z * hh
    out_ref[0] = h.reshape(nb, n, d)


def _run(inputs, supports, wglc, wmsg, bmsg, wrzh, urz, uh, interpret=False):
    seq, b, n, d = inputs.shape
    hd = NUM_HEADS * DH
    i = jax.lax.broadcasted_iota(jnp.int32, (hd, hd), 0) // DH
    j = jax.lax.broadcasted_iota(jnp.int32, (hd, hd), 1) // DH
    hmask = (i == j).astype(jnp.bfloat16)

    out, adj = pl.pallas_call(
        _sglc_body,
        grid=(seq,),
        in_specs=[
            pl.BlockSpec((1, b, n, d), lambda ti: (ti, 0, 0, 0)),
            pl.BlockSpec((b, n, n), lambda ti: (0, 0, 0)),
            pl.BlockSpec((d, hd), lambda ti: (0, 0)),
            pl.BlockSpec((d, d), lambda ti: (0, 0)),
            pl.BlockSpec((1, d), lambda ti: (0, 0)),
            pl.BlockSpec((d, 3 * d), lambda ti: (0, 0)),
            pl.BlockSpec((d, 2 * d), lambda ti: (0, 0)),
            pl.BlockSpec((d, d), lambda ti: (0, 0)),
            pl.BlockSpec((hd, hd), lambda ti: (0, 0)),
        ],
        out_specs=[
            pl.BlockSpec((1, b, n, d), lambda ti: (ti, 0, 0, 0)),
            pl.BlockSpec((b, n, n), lambda ti: (0, 0, 0)),
        ],
        out_shape=[
            jax.ShapeDtypeStruct((seq, b, n, d), jnp.float32),
            jax.ShapeDtypeStruct((b, n, n), jnp.float32),
        ],
        interpret=interpret,
    )(inputs, supports, wglc, wmsg, bmsg, wrzh, urz, uh, hmask)
    return out, adj


def kernel(inputs, supports, W_gl, W_msg, b_msg, Wr, Ur, Wz, Uz, Wh, Uh):
    d = inputs.shape[-1]
    b = inputs.shape[1]
    bf = jnp.bfloat16
    wglc = jnp.transpose(W_gl[0], (1, 0, 2)).reshape(d, NUM_HEADS * DH)
    wrzh = jnp.concatenate([Wr[0], Wz[0], Wh[0]], axis=1)
    urz = jnp.concatenate([Ur[0], Uz[0]], axis=1)
    args = (inputs, supports, wglc.astype(bf), W_msg[0].astype(bf),
            b_msg[0].reshape(1, d), wrzh.astype(bf), urz.astype(bf),
            Uh[0].astype(bf))
    devs = jax.devices()
    nsh = 1
    for c in (4, 2):
        if len(devs) >= c and b % c == 0:
            nsh = c
            break
    if nsh == 1:
        return _run(*args)
    mesh = Mesh(np.array(devs[:nsh]), ("b",))
    fn = shard_map(
        _run, mesh=mesh,
        in_specs=(P(None, "b"), P("b"), P(), P(), P(), P(), P(), P()),
        out_specs=(P(None, "b"), P("b")), check_rep=False)
    return fn(*args)


# speedup vs baseline: 2.2797x; 2.2797x over previous
import jax
import jax.numpy as jnp
from jax.experimental import pallas as pl

NUM_HEADS = 4
NUM_STEPS = 3
SKIP = 0.3
DH = 32


def _dotT(a, b):
    return jax.lax.dot_general(
        a, b, (((1,), (1,)), ((), ())), preferred_element_type=jnp.float32
    )


def _dot(a, b):
    return jnp.dot(a, b, preferred_element_type=jnp.float32)


def _sglc_body(x_ref, sup_ref, wgl_ref, wmsg_ref, bmsg_ref, wr_ref, ur_ref,
               wz_ref, uz_ref, wh_ref, uh_ref, out_ref, adj_ref):
    t = pl.program_id(0)
    nb, n, d = x_ref.shape[1], x_ref.shape[2], x_ref.shape[3]
    bf = jnp.bfloat16
    hd = NUM_HEADS * DH

    wglc = jnp.concatenate([wgl_ref[0, h] for h in range(NUM_HEADS)],
                           axis=1).astype(bf)
    ii = jax.lax.broadcasted_iota(jnp.int32, (hd, hd), 0) // DH
    jj = jax.lax.broadcasted_iota(jnp.int32, (hd, hd), 1) // DH
    hmask = (ii == jj).astype(bf)
    wmsg = wmsg_ref[0].astype(bf)
    bmsg = bmsg_ref[...]
    wrzh = jnp.concatenate([wr_ref[0], wz_ref[0], wh_ref[0]],
                           axis=1).astype(bf)
    urz = jnp.concatenate([ur_ref[0], uz_ref[0]], axis=1).astype(bf)
    uh = uh_ref[0].astype(bf)

    x_all = x_ref[0].reshape(nb * n, d)

    xp = _dot(x_all.astype(bf), wglc)
    sq = (xp * xp).astype(bf)
    s = _dot(sq, hmask)
    xn = (xp * (0.5 / (jnp.sqrt(s) + 1e-8))).astype(bf)

    @pl.when(t == 0)
    def _init():
        adj_ref[...] = sup_ref[...]

    adjs = []
    for b in range(nb):
        xnb = xn[b * n:(b + 1) * n]
        attn = jnp.maximum(_dotT(xnb, xnb), 0.0)
        learned = attn / (jnp.sum(attn, axis=-1, keepdims=True) + 1e-8)
        adj = SKIP * adj_ref[b] + (1.0 - SKIP) * learned
        adj_ref[b] = adj
        adjs.append(adj.astype(bf))

    h = x_all
    for _ in range(NUM_STEPS):
        hb = h.astype(bf)
        a_all = jnp.concatenate(
            [_dot(adjs[b], hb[b * n:(b + 1) * n]) for b in range(nb)], axis=0)
        m = (_dot(a_all.astype(bf), wmsg) + bmsg).astype(bf)
        gates = _dot(m, wrzh)
        hu = _dot(hb, urz)
        r = jax.nn.sigmoid(gates[:, :d] + hu[:, :d])
        z = jax.nn.sigmoid(gates[:, d:2 * d] + hu[:, d:2 * d])
        q = _dot((r * h).astype(bf), uh)
        hh = jnp.tanh(gates[:, 2 * d:] + q)
        h = (1.0 - z) * h + z * hh
    out_ref[0] = h.reshape(nb, n, d)


def _run(inputs, supports, W_gl, W_msg, b_msg, Wr, Ur, Wz, Uz, Wh, Uh,
         interpret=False):
    seq, b, n, d = inputs.shape

    def _w(spec_shape):
        return pl.BlockSpec(spec_shape, lambda ti: (0,) * len(spec_shape))

    out, adj = pl.pallas_call(
        _sglc_body,
        grid=(seq,),
        in_specs=[
            pl.BlockSpec((1, b, n, d), lambda ti: (ti, 0, 0, 0)),
            pl.BlockSpec((b, n, n), lambda ti: (0, 0, 0)),
            _w((1, NUM_HEADS, d, DH)),
            _w((1, d, d)),
            _w((1, d)),
            _w((1, d, d)),
            _w((1, d, d)),
            _w((1, d, d)),
            _w((1, d, d)),
            _w((1, d, d)),
            _w((1, d, d)),
        ],
        out_specs=[
            pl.BlockSpec((1, b, n, d), lambda ti: (ti, 0, 0, 0)),
            pl.BlockSpec((b, n, n), lambda ti: (0, 0, 0)),
        ],
        out_shape=[
            jax.ShapeDtypeStruct((seq, b, n, d), jnp.float32),
            jax.ShapeDtypeStruct((b, n, n), jnp.float32),
        ],
        interpret=interpret,
    )(inputs, supports, W_gl, W_msg, b_msg, Wr, Ur, Wz, Uz, Wh, Uh)
    return out, adj


def kernel(inputs, supports, W_gl, W_msg, b_msg, Wr, Ur, Wz, Uz, Wh, Uh):
    return _run(inputs, supports, W_gl, W_msg, b_msg, Wr, Ur, Wz, Uz, Wh, Uh)


# speedup vs baseline: 2.4277x; 1.0649x over previous
import jax
import jax.numpy as jnp
from jax.experimental import pallas as pl

NUM_HEADS = 4
NUM_STEPS = 3
SKIP = 0.3
DH = 32
T_UNROLL = 2


def _dotT(a, b):
    return jax.lax.dot_general(
        a, b, (((1,), (1,)), ((), ())), preferred_element_type=jnp.float32
    )


def _dot(a, b):
    return jnp.dot(a, b, preferred_element_type=jnp.float32)


def _sglc_body(x_ref, sup_ref, wgl_ref, wmsg_ref, bmsg_ref, wr_ref, ur_ref,
               wz_ref, uz_ref, wh_ref, uh_ref, out_ref, adj_ref):
    t = pl.program_id(0)
    nb, n, d = x_ref.shape[1], x_ref.shape[2], x_ref.shape[3]
    bf = jnp.bfloat16
    hd = NUM_HEADS * DH

    wglc = jnp.concatenate([wgl_ref[0, h] for h in range(NUM_HEADS)],
                           axis=1).astype(bf)
    ii = jax.lax.broadcasted_iota(jnp.int32, (hd, hd), 0) // DH
    jj = jax.lax.broadcasted_iota(jnp.int32, (hd, hd), 1) // DH
    hmask = (ii == jj).astype(bf)
    wmsg = wmsg_ref[0].astype(bf)
    bmsg = bmsg_ref[...]
    wrzh = jnp.concatenate([wr_ref[0], wz_ref[0], wh_ref[0]],
                           axis=1).astype(bf)
    urz = jnp.concatenate([ur_ref[0], uz_ref[0]], axis=1).astype(bf)
    uh = uh_ref[0].astype(bf)

    @pl.when(t == 0)
    def _init():
        adj_ref[...] = sup_ref[...]

    for k in range(T_UNROLL):
        x_all = x_ref[k].reshape(nb * n, d)

        xp = _dot(x_all.astype(bf), wglc)
        sq = (xp * xp).astype(bf)
        s = _dot(sq, hmask)
        xn = (xp * (0.5 / (jnp.sqrt(s) + 1e-8))).astype(bf)

        adjs = []
        for b in range(nb):
            xnb = xn[b * n:(b + 1) * n]
            attn = jnp.maximum(_dotT(xnb, xnb), 0.0)
            learned = attn / (jnp.sum(attn, axis=-1, keepdims=True) + 1e-8)
            adj = SKIP * adj_ref[b] + (1.0 - SKIP) * learned
            adj_ref[b] = adj
            adjs.append(adj.astype(bf))

        h = x_all
        for _ in range(NUM_STEPS):
            hb = h.astype(bf)
            a_all = jnp.concatenate(
                [_dot(adjs[b], hb[b * n:(b + 1) * n]) for b in range(nb)],
                axis=0)
            m = (_dot(a_all.astype(bf), wmsg) + bmsg).astype(bf)
            gates = _dot(m, wrzh)
            hu = _dot(hb, urz)
            r = jax.nn.sigmoid(gates[:, :d] + hu[:, :d])
            z = jax.nn.sigmoid(gates[:, d:2 * d] + hu[:, d:2 * d])
            q = _dot((r * h).astype(bf), uh)
            hh = jnp.tanh(gates[:, 2 * d:] + q)
            h = (1.0 - z) * h + z * hh
        out_ref[k] = h.reshape(nb, n, d)


def _run(inputs, supports, W_gl, W_msg, b_msg, Wr, Ur, Wz, Uz, Wh, Uh,
         interpret=False):
    seq, b, n, d = inputs.shape

    def _w(spec_shape):
        return pl.BlockSpec(spec_shape, lambda ti: (0,) * len(spec_shape))

    out, adj = pl.pallas_call(
        _sglc_body,
        grid=(seq // T_UNROLL,),
        in_specs=[
            pl.BlockSpec((T_UNROLL, b, n, d), lambda ti: (ti, 0, 0, 0)),
            pl.BlockSpec((b, n, n), lambda ti: (0, 0, 0)),
            _w((1, NUM_HEADS, d, DH)),
            _w((1, d, d)),
            _w((1, d)),
            _w((1, d, d)),
            _w((1, d, d)),
            _w((1, d, d)),
            _w((1, d, d)),
            _w((1, d, d)),
            _w((1, d, d)),
        ],
        out_specs=[
            pl.BlockSpec((T_UNROLL, b, n, d), lambda ti: (ti, 0, 0, 0)),
            pl.BlockSpec((b, n, n), lambda ti: (0, 0, 0)),
        ],
        out_shape=[
            jax.ShapeDtypeStruct((seq, b, n, d), jnp.float32),
            jax.ShapeDtypeStruct((b, n, n), jnp.float32),
        ],
        interpret=interpret,
    )(inputs, supports, W_gl, W_msg, b_msg, Wr, Ur, Wz, Uz, Wh, Uh)
    return out, adj


def kernel(inputs, supports, W_gl, W_msg, b_msg, Wr, Ur, Wz, Uz, Wh, Uh):
    return _run(inputs, supports, W_gl, W_msg, b_msg, Wr, Ur, Wz, Uz, Wh, Uh)
